# Initial kernel scaffold; baseline (speedup 1.0000x reference)
import jax
import jax.numpy as jnp
from jax.experimental import pallas as pl


def kernel(x, edge_index, edge_attr, params):
    raise NotImplementedError("write your pallas kernel here")


# speedup vs baseline: 10.4311x; 10.4311x over previous
import functools

import jax
import jax.numpy as jnp
from jax import lax
from jax.experimental import pallas as pl
from jax.experimental.pallas import tpu as pltpu
from jax.experimental.pallas import tpu_sc as plsc

N = 10000
E = 320000
IN_DIM = 128
HID = 192
HEADS = 12
OUT = 16
LAYERS = 3
NEG_SLOPE = 0.2
EPS = 1e-5

NW = 32
DPW = 320
NPAD = NW * DPW
C = 64
E_PAD = E + C

R = 1000
_INV_SQRT2 = 0.7071067811865476


def _ln(h, g, b):
    mu = jnp.mean(h, axis=-1, keepdims=True)
    var = jnp.mean((h - mu) ** 2, axis=-1, keepdims=True)
    return (h - mu) / jnp.sqrt(var + EPS) * g + b


def _tc_input_body(x_ref, w_ref, b_ref, g_ref, be_ref, o_ref):
    h = jnp.dot(x_ref[...], w_ref[...], preferred_element_type=jnp.float32)
    h = h + b_ref[...]
    h = 0.5 * h * (1.0 + lax.erf(h * _INV_SQRT2))
    o_ref[...] = _ln(h, g_ref[...], be_ref[...])


def _tc_pre_body(h_ref, wl_ref, bl_ref, wr_ref, br_ref, xl_ref, xr_ref):
    h = h_ref[...]
    xl_ref[...] = jnp.dot(h, wl_ref[...], preferred_element_type=jnp.float32) + bl_ref[...]
    xr_ref[...] = jnp.dot(h, wr_ref[...], preferred_element_type=jnp.float32) + br_ref[...]


def _tc_post_body(h_ref, m_ref, bo_ref, g_ref, b_ref, o_ref):
    o_ref[...] = _ln(h_ref[...] + m_ref[...] + bo_ref[...], g_ref[...], b_ref[...])


def _tc_final_body(h_ref, wg_ref, bg_ref, g_ref, b_ref, o_ref):
    h = h_ref[...]
    z = jnp.dot(h, wg_ref[...], preferred_element_type=jnp.float32) + bg_ref[...]
    gate = 1.0 / (1.0 + jnp.exp(-z))
    o_ref[...] = _ln(h * gate, g_ref[...], b_ref[...])


def _row_spec(d):
    return pl.BlockSpec((R, d), lambda i: (i, 0))


def _full_spec(shape):
    return pl.BlockSpec(shape, lambda i: tuple(0 for _ in shape))


def _tc_input(x, w, b, g, be):
    return pl.pallas_call(
        _tc_input_body,
        grid=(N // R,),
        in_specs=[_row_spec(IN_DIM), _full_spec((IN_DIM, HID)),
                  _full_spec((1, HID)), _full_spec((1, HID)), _full_spec((1, HID))],
        out_specs=_row_spec(HID),
        out_shape=jax.ShapeDtypeStruct((N, HID), jnp.float32),
    )(x, w, b, g, be)


def _tc_pre(h, wl, bl, wr, br):
    return pl.pallas_call(
        _tc_pre_body,
        grid=(N // R,),
        in_specs=[_row_spec(HID), _full_spec((HID, HID)), _full_spec((1, HID)),
                  _full_spec((HID, HID)), _full_spec((1, HID))],
        out_specs=[_row_spec(HID), _row_spec(HID)],
        out_shape=[jax.ShapeDtypeStruct((N, HID), jnp.float32),
                   jax.ShapeDtypeStruct((N, HID), jnp.float32)],
    )(h, wl, bl, wr, br)


def _tc_post(h, m, bo, g, b):
    return pl.pallas_call(
        _tc_post_body,
        grid=(N // R,),
        in_specs=[_row_spec(HID), _row_spec(HID), _full_spec((1, HID)),
                  _full_spec((1, HID)), _full_spec((1, HID))],
        out_specs=_row_spec(HID),
        out_shape=jax.ShapeDtypeStruct((N, HID), jnp.float32),
    )(h, m, bo, g, b)


def _tc_final(h, wg, bg, g, b):
    return pl.pallas_call(
        _tc_final_body,
        grid=(N // R,),
        in_specs=[_row_spec(HID), _full_spec((HID, HID)), _full_spec((1, HID)),
                  _full_spec((1, HID)), _full_spec((1, HID))],
        out_specs=_row_spec(HID),
        out_shape=jax.ShapeDtypeStruct((N, HID), jnp.float32),
    )(h, wg, bg, g, b)


_mesh = plsc.VectorSubcoreMesh(core_axis_name="c", subcore_axis_name="s")

_GDN = lax.GatherDimensionNumbers(
    offset_dims=(), collapsed_slice_dims=(0,), start_index_map=(0,))


def _shuf(x, idx):
    return lax.gather(x, idx[:, None], _GDN, (1,),
                      mode=lax.GatherScatterMode.PROMISE_IN_BOUNDS)


@functools.partial(
    pl.kernel,
    out_type=jax.ShapeDtypeStruct((NPAD * HID,), jnp.float32),
    mesh=_mesh,
    compiler_params=pltpu.CompilerParams(use_tc_tiling_on_sc=False),
    scratch_types=[
        pltpu.VMEM((C,), jnp.int32),
        pltpu.VMEM((C + 16,), jnp.int32),
        pltpu.VMEM((C,), jnp.float32),
        pltpu.VMEM((C,), jnp.float32),
        pltpu.VMEM((C, HID), jnp.float32),
        pltpu.VMEM((C, HID), jnp.float32),
        pltpu.VMEM(((DPW + 1) * HID,), jnp.float32),
        pltpu.VMEM(((DPW + 1) * 16,), jnp.float32),
        pltpu.VMEM((2, HID), jnp.float32),
        pltpu.VMEM((16, 16), jnp.float32),
        pltpu.VMEM((48,), jnp.int32),
        pltpu.VMEM((48,), jnp.int32),
        pltpu.SemaphoreType.DMA,
        pltpu.SemaphoreType.DMA,
    ],
)
def _sc_gat(xl_hbm, xr_hbm, src_hbm, dst_hbm, ea0_hbm, ea1_hbm, barr_hbm,
            earr_hbm, we_hbm, att_hbm, out_hbm,
            idxs_v, idxd_v, ea0_v, ea1_v, xl_v, xr_v, num_v, den_v,
            we_v, att_v, barr_v, earr_v, sem1, sem2):
    wid = lax.axis_index("s") * 2 + lax.axis_index("c")
    base = wid * DPW

    pltpu.sync_copy(barr_hbm, barr_v.at[pl.ds(0, NW)])
    pltpu.sync_copy(earr_hbm, earr_v.at[pl.ds(0, NW)])
    pltpu.sync_copy(we_hbm, we_v)
    pltpu.sync_copy(att_hbm, att_v)

    lane = lax.iota(jnp.int32, 16)
    zero16 = jnp.zeros((16,), jnp.float32)
    b_w = barr_v[pl.ds(wid, 16)][0]
    e_w = earr_v[pl.ds(wid, 16)][0]
    start0 = b_w & jnp.int32(-C)
    nch = jnp.right_shift(e_w - start0 + jnp.int32(C - 1), 6)

    def _zero_num(i, carry):
        num_v[pl.ds(pl.multiple_of(i * 16, 16), 16)] = zero16
        return carry

    lax.fori_loop(0, (DPW + 1) * HID // 16, _zero_num, 0)

    def _zero_den(i, carry):
        den_v[pl.ds(pl.multiple_of(i * 16, 16), 16)] = zero16
        return carry

    lax.fori_loop(0, DPW + 1, _zero_den, 0)

    def _edge(e, ecarry):
        g16 = pl.multiple_of((e >> 4) << 4, 16)
        jj = jnp.full((16,), e & 15, jnp.int32)
        d = idxd_v[pl.ds(e, 16)][0]
        own = (d >= base) & (d < base + DPW)
        dl = jnp.where(own, d - base, jnp.int32(DPW))
        a0v = _shuf(ea0_v[pl.ds(g16, 16)], jj)
        a1v = _shuf(ea1_v[pl.ds(g16, 16)], jj)
        rb = dl * HID
        dvec = zero16
        for h in range(HEADS):
            xlh = xl_v[e, pl.ds(16 * h, 16)]
            xrh = xr_v[e, pl.ds(16 * h, 16)]
            sh = xlh + xrh + a0v * we_v[0, pl.ds(16 * h, 16)] \
                + a1v * we_v[1, pl.ds(16 * h, 16)]
            sh = jnp.where(sh >= 0.0, sh, NEG_SLOPE * sh)
            prod = sh * att_v[h]
            for dd in (8, 4, 2, 1):
                prod = prod + _shuf(prod, lane ^ dd)
            av = jnp.exp(prod)
            plsc.addupdate(
                num_v.at[pl.ds(pl.multiple_of(rb + 16 * h, 16), 16)], av * xlh)
            dvec = dvec + jnp.where(lane == h, av, zero16)
        plsc.addupdate(den_v.at[pl.ds(pl.multiple_of(dl * 16, 16), 16)], dvec)
        return ecarry

    @pl.loop(0, nch)
    def _chunk(k):
        s = pl.multiple_of(start0 + k * C, C)
        pltpu.sync_copy(src_hbm.at[pl.ds(s, C)], idxs_v)
        pltpu.sync_copy(dst_hbm.at[pl.ds(s, C)], idxd_v.at[pl.ds(0, C)])
        pltpu.sync_copy(ea0_hbm.at[pl.ds(s, C)], ea0_v)
        pltpu.sync_copy(ea1_hbm.at[pl.ds(s, C)], ea1_v)
        gl = pltpu.async_copy(xl_hbm.at[idxs_v], xl_v, sem1)
        gr = pltpu.async_copy(xr_hbm.at[idxd_v.at[pl.ds(0, C)]], xr_v, sem2)
        gl.wait()
        gr.wait()
        lax.fori_loop(0, C, _edge, 0)

    def _fin(d, carry):
        dv = den_v[pl.ds(pl.multiple_of(d * 16, 16), 16)]
        inv = 1.0 / (dv + 1e-16)
        for h in range(HEADS):
            off = pl.multiple_of(d * HID + 16 * h, 16)
            num_v[pl.ds(off, 16)] = num_v[pl.ds(off, 16)] * _shuf(inv, jnp.full((16,), h, jnp.int32))
        return carry

    lax.fori_loop(0, DPW, _fin, 0)

    pltpu.sync_copy(
        num_v.at[pl.ds(0, DPW * HID)],
        out_hbm.at[pl.ds(pl.multiple_of(base * HID, 64), DPW * HID)])


def kernel(x, edge_index, edge_attr, params):
    p = params
    src = edge_index[0]
    dst = edge_index[1]

    order = jnp.argsort(dst)
    src_s = jnp.take(src, order)
    dst_s = jnp.take(dst, order)
    ea_s = jnp.take(edge_attr, order, axis=0)
    src_p = jnp.concatenate([src_s, jnp.zeros((C,), jnp.int32)])
    dst_p = jnp.concatenate([dst_s, jnp.full((C,), NPAD, jnp.int32)])
    ea0_p = jnp.concatenate([ea_s[:, 0], jnp.zeros((C,), jnp.float32)])
    ea1_p = jnp.concatenate([ea_s[:, 1], jnp.zeros((C,), jnp.float32)])
    marks = jnp.arange(0, NPAD + 1, DPW, dtype=jnp.int32)
    bnd = jnp.searchsorted(dst_s, marks).astype(jnp.int32)
    barr = bnd[:NW]
    earr = bnd[1:NW + 1]

    def row(v):
        return v.reshape(1, HID)

    h = _tc_input(x, p['W_in'], row(p['b_in']), row(p['g_in']), row(p['be_in']))
    for l in range(LAYERS):
        xl, xr = _tc_pre(h, p[f'Wl{l}'], row(p[f'bl{l}']), p[f'Wr{l}'], row(p[f'br{l}']))
        att_pad = jnp.zeros((16, 16), jnp.float32).at[:HEADS].set(p[f'att{l}'])
        msg = _sc_gat(xl, xr, src_p, dst_p, ea0_p, ea1_p, barr, earr,
                      p[f'We{l}'], att_pad)
        msg = msg.reshape(NPAD, HID)[:N]
        h = _tc_post(h, msg, row(p[f'bo{l}']), row(p[f'g{l}']), row(p[f'b{l}']))
    return _tc_final(h, p['Wg'], row(p['bg']), row(p['g_f']), row(p['b_f']))


# speedup vs baseline: 26.6153x; 2.5515x over previous
import functools

import jax
import jax.numpy as jnp
from jax import lax
from jax.experimental import pallas as pl
from jax.experimental.pallas import tpu as pltpu
from jax.experimental.pallas import tpu_sc as plsc

N = 10000
E = 320000
IN_DIM = 128
HID = 192
HEADS = 12
OUT = 16
LAYERS = 3
NEG_SLOPE = 0.2
EPS = 1e-5

NW = 32
DPW = 320
NPAD = NW * DPW
C = 64
E_PAD = E + C

R = 1000
_INV_SQRT2 = 0.7071067811865476


def _ln(h, g, b):
    mu = jnp.mean(h, axis=-1, keepdims=True)
    var = jnp.mean((h - mu) ** 2, axis=-1, keepdims=True)
    return (h - mu) / jnp.sqrt(var + EPS) * g + b


def _tc_input_body(x_ref, w_ref, b_ref, g_ref, be_ref, o_ref):
    h = jnp.dot(x_ref[...], w_ref[...], preferred_element_type=jnp.float32)
    h = h + b_ref[...]
    h = 0.5 * h * (1.0 + lax.erf(h * _INV_SQRT2))
    o_ref[...] = _ln(h, g_ref[...], be_ref[...])


def _tc_pre_body(h_ref, wl_ref, bl_ref, wr_ref, br_ref, xl_ref, xr_ref):
    h = h_ref[...]
    xl_ref[...] = jnp.dot(h, wl_ref[...], preferred_element_type=jnp.float32) + bl_ref[...]
    xr_ref[...] = jnp.dot(h, wr_ref[...], preferred_element_type=jnp.float32) + br_ref[...]


def _tc_post_body(h_ref, m_ref, bo_ref, g_ref, b_ref, o_ref):
    o_ref[...] = _ln(h_ref[...] + m_ref[...] + bo_ref[...], g_ref[...], b_ref[...])


def _tc_final_body(h_ref, wg_ref, bg_ref, g_ref, b_ref, o_ref):
    h = h_ref[...]
    z = jnp.dot(h, wg_ref[...], preferred_element_type=jnp.float32) + bg_ref[...]
    gate = 1.0 / (1.0 + jnp.exp(-z))
    o_ref[...] = _ln(h * gate, g_ref[...], b_ref[...])


def _row_spec(d):
    return pl.BlockSpec((R, d), lambda i: (i, 0))


def _full_spec(shape):
    return pl.BlockSpec(shape, lambda i: tuple(0 for _ in shape))


def _tc_input(x, w, b, g, be):
    return pl.pallas_call(
        _tc_input_body,
        grid=(N // R,),
        in_specs=[_row_spec(IN_DIM), _full_spec((IN_DIM, HID)),
                  _full_spec((1, HID)), _full_spec((1, HID)), _full_spec((1, HID))],
        out_specs=_row_spec(HID),
        out_shape=jax.ShapeDtypeStruct((N, HID), jnp.float32),
    )(x, w, b, g, be)


def _tc_pre(h, wl, bl, wr, br):
    return pl.pallas_call(
        _tc_pre_body,
        grid=(N // R,),
        in_specs=[_row_spec(HID), _full_spec((HID, HID)), _full_spec((1, HID)),
                  _full_spec((HID, HID)), _full_spec((1, HID))],
        out_specs=[_row_spec(HID), _row_spec(HID)],
        out_shape=[jax.ShapeDtypeStruct((N, HID), jnp.float32),
                   jax.ShapeDtypeStruct((N, HID), jnp.float32)],
    )(h, wl, bl, wr, br)


def _tc_post(h, m, bo, g, b):
    return pl.pallas_call(
        _tc_post_body,
        grid=(N // R,),
        in_specs=[_row_spec(HID), _row_spec(HID), _full_spec((1, HID)),
                  _full_spec((1, HID)), _full_spec((1, HID))],
        out_specs=_row_spec(HID),
        out_shape=jax.ShapeDtypeStruct((N, HID), jnp.float32),
    )(h, m, bo, g, b)


def _tc_final(h, wg, bg, g, b):
    return pl.pallas_call(
        _tc_final_body,
        grid=(N // R,),
        in_specs=[_row_spec(HID), _full_spec((HID, HID)), _full_spec((1, HID)),
                  _full_spec((1, HID)), _full_spec((1, HID))],
        out_specs=_row_spec(HID),
        out_shape=jax.ShapeDtypeStruct((N, HID), jnp.float32),
    )(h, wg, bg, g, b)


_mesh = plsc.VectorSubcoreMesh(core_axis_name="c", subcore_axis_name="s")

_GDN = lax.GatherDimensionNumbers(
    offset_dims=(), collapsed_slice_dims=(0,), start_index_map=(0,))


def _shuf(x, idx):
    return lax.gather(x, idx[:, None], _GDN, (1,),
                      mode=lax.GatherScatterMode.PROMISE_IN_BOUNDS)


@functools.partial(
    pl.kernel,
    out_type=jax.ShapeDtypeStruct((NPAD * HID,), jnp.float32),
    mesh=_mesh,
    compiler_params=pltpu.CompilerParams(use_tc_tiling_on_sc=False),
    scratch_types=[
        pltpu.VMEM((C,), jnp.int32),
        pltpu.VMEM((C + 16,), jnp.int32),
        pltpu.VMEM((C,), jnp.float32),
        pltpu.VMEM((C,), jnp.float32),
        pltpu.VMEM((C, HID), jnp.float32),
        pltpu.VMEM((C, HID), jnp.float32),
        pltpu.VMEM(((DPW + 1) * HID,), jnp.float32),
        pltpu.VMEM(((DPW + 1) * 16,), jnp.float32),
        pltpu.VMEM((2, HID), jnp.float32),
        pltpu.VMEM((16, 16), jnp.float32),
        pltpu.VMEM((48,), jnp.int32),
        pltpu.VMEM((48,), jnp.int32),
        pltpu.SemaphoreType.DMA,
        pltpu.SemaphoreType.DMA,
    ],
)
def _sc_gat(xl_hbm, xr_hbm, src_hbm, dst_hbm, ea0_hbm, ea1_hbm, barr_hbm,
            earr_hbm, we_hbm, att_hbm, out_hbm,
            idxs_v, idxd_v, ea0_v, ea1_v, xl_v, xr_v, num_v, den_v,
            we_v, att_v, barr_v, earr_v, sem1, sem2):
    wid = lax.axis_index("s") * 2 + lax.axis_index("c")
    base = wid * DPW

    pltpu.sync_copy(barr_hbm, barr_v.at[pl.ds(0, NW)])
    pltpu.sync_copy(earr_hbm, earr_v.at[pl.ds(0, NW)])
    pltpu.sync_copy(we_hbm, we_v)
    pltpu.sync_copy(att_hbm, att_v)

    lane = lax.iota(jnp.int32, 16)
    zero16 = jnp.zeros((16,), jnp.float32)
    b_w = barr_v[pl.ds(wid, 16)][0]
    e_w = earr_v[pl.ds(wid, 16)][0]
    start0 = b_w & jnp.int32(-C)
    nch = jnp.right_shift(e_w - start0 + jnp.int32(C - 1), 6)

    def _zero_num(i, carry):
        num_v[pl.ds(pl.multiple_of(i * 16, 16), 16)] = zero16
        return carry

    lax.fori_loop(0, (DPW + 1) * HID // 16, _zero_num, 0)

    def _zero_den(i, carry):
        den_v[pl.ds(pl.multiple_of(i * 16, 16), 16)] = zero16
        return carry

    lax.fori_loop(0, DPW + 1, _zero_den, 0)

    we0 = [we_v[0, pl.ds(16 * h, 16)] for h in range(HEADS)]
    we1 = [we_v[1, pl.ds(16 * h, 16)] for h in range(HEADS)]
    att_r = [att_v[h] for h in range(HEADS)]
    bitrev = (((lane & 1) << 3) | ((lane & 2) << 1)
              | ((lane & 4) >> 1) | ((lane & 8) >> 3))

    def _combine(a, b, d):
        m = (lane & d) == 0
        return jnp.where(m, a, b) + _shuf(jnp.where(m, b, a), lane ^ d)

    def _edge(e, ecarry):
        g16 = pl.multiple_of((e >> 4) << 4, 16)
        jj = jnp.full((16,), e & 15, jnp.int32)
        d = idxd_v[pl.ds(e, 16)][0]
        own = (d >= base) & (d < base + DPW)
        dl = jnp.where(own, d - base, jnp.int32(DPW))
        a0v = _shuf(ea0_v[pl.ds(g16, 16)], jj)
        a1v = _shuf(ea1_v[pl.ds(g16, 16)], jj)
        rb = dl * HID
        xls = []
        vecs = []
        for h in range(HEADS):
            xlh = xl_v[e, pl.ds(16 * h, 16)]
            xrh = xr_v[e, pl.ds(16 * h, 16)]
            sh = xlh + xrh + a0v * we0[h] + a1v * we1[h]
            sh = jnp.where(sh >= 0.0, sh, NEG_SLOPE * sh)
            vecs.append(sh * att_r[h])
            xls.append(xlh)
        vecs += [zero16] * (16 - HEADS)
        for dd in (8, 4, 2, 1):
            vecs = [_combine(vecs[i], vecs[i + 1], dd)
                    for i in range(0, len(vecs), 2)]
        av = jnp.exp(_shuf(vecs[0], bitrev))
        for h in range(HEADS):
            avh = _shuf(av, jnp.full((16,), h, jnp.int32))
            plsc.addupdate(
                num_v.at[pl.ds(pl.multiple_of(rb + 16 * h, 16), 16)], avh * xls[h])
        plsc.addupdate(den_v.at[pl.ds(pl.multiple_of(dl * 16, 16), 16)], av)
        return ecarry

    @pl.loop(0, nch)
    def _chunk(k):
        s = pl.multiple_of(start0 + k * C, C)
        pltpu.sync_copy(src_hbm.at[pl.ds(s, C)], idxs_v)
        pltpu.sync_copy(dst_hbm.at[pl.ds(s, C)], idxd_v.at[pl.ds(0, C)])
        pltpu.sync_copy(ea0_hbm.at[pl.ds(s, C)], ea0_v)
        pltpu.sync_copy(ea1_hbm.at[pl.ds(s, C)], ea1_v)
        gl = pltpu.async_copy(xl_hbm.at[idxs_v], xl_v, sem1)
        gr = pltpu.async_copy(xr_hbm.at[idxd_v.at[pl.ds(0, C)]], xr_v, sem2)
        gl.wait()
        gr.wait()
        lax.fori_loop(0, C, _edge, 0)

    def _fin(d, carry):
        dv = den_v[pl.ds(pl.multiple_of(d * 16, 16), 16)]
        inv = 1.0 / (dv + 1e-16)
        for h in range(HEADS):
            off = pl.multiple_of(d * HID + 16 * h, 16)
            num_v[pl.ds(off, 16)] = num_v[pl.ds(off, 16)] * _shuf(inv, jnp.full((16,), h, jnp.int32))
        return carry

    lax.fori_loop(0, DPW, _fin, 0)

    pltpu.sync_copy(
        num_v.at[pl.ds(0, DPW * HID)],
        out_hbm.at[pl.ds(pl.multiple_of(base * HID, 64), DPW * HID)])


def kernel(x, edge_index, edge_attr, params):
    p = params
    src = edge_index[0]
    dst = edge_index[1]

    order = jnp.argsort(dst)
    src_s = jnp.take(src, order)
    dst_s = jnp.take(dst, order)
    ea_s = jnp.take(edge_attr, order, axis=0)
    src_p = jnp.concatenate([src_s, jnp.zeros((C,), jnp.int32)])
    dst_p = jnp.concatenate([dst_s, jnp.full((C,), NPAD, jnp.int32)])
    ea0_p = jnp.concatenate([ea_s[:, 0], jnp.zeros((C,), jnp.float32)])
    ea1_p = jnp.concatenate([ea_s[:, 1], jnp.zeros((C,), jnp.float32)])
    marks = jnp.arange(0, NPAD + 1, DPW, dtype=jnp.int32)
    bnd = jnp.searchsorted(dst_s, marks).astype(jnp.int32)
    barr = bnd[:NW]
    earr = bnd[1:NW + 1]

    def row(v):
        return v.reshape(1, HID)

    h = _tc_input(x, p['W_in'], row(p['b_in']), row(p['g_in']), row(p['be_in']))
    for l in range(LAYERS):
        xl, xr = _tc_pre(h, p[f'Wl{l}'], row(p[f'bl{l}']), p[f'Wr{l}'], row(p[f'br{l}']))
        att_pad = jnp.zeros((16, 16), jnp.float32).at[:HEADS].set(p[f'att{l}'])
        msg = _sc_gat(xl, xr, src_p, dst_p, ea0_p, ea1_p, barr, earr,
                      p[f'We{l}'], att_pad)
        msg = msg.reshape(NPAD, HID)[:N]
        h = _tc_post(h, msg, row(p[f'bo{l}']), row(p[f'g{l}']), row(p[f'b{l}']))
    return _tc_final(h, p['Wg'], row(p['bg']), row(p['g_f']), row(p['b_f']))


# speedup vs baseline: 30.8880x; 1.1605x over previous
import functools

import jax
import jax.numpy as jnp
from jax import lax
from jax.experimental import pallas as pl
from jax.experimental.pallas import tpu as pltpu
from jax.experimental.pallas import tpu_sc as plsc

N = 10000
E = 320000
IN_DIM = 128
HID = 192
HEADS = 12
OUT = 16
LAYERS = 3
NEG_SLOPE = 0.2
EPS = 1e-5

NW = 32
NR = 64
DPR = 160
NPAD = NR * DPR
C = 64
E_PAD = E + C

R = 1000
_INV_SQRT2 = 0.7071067811865476


def _ln(h, g, b):
    mu = jnp.mean(h, axis=-1, keepdims=True)
    var = jnp.mean((h - mu) ** 2, axis=-1, keepdims=True)
    return (h - mu) / jnp.sqrt(var + EPS) * g + b


def _tc_input_body(x_ref, w_ref, b_ref, g_ref, be_ref, o_ref):
    h = jnp.dot(x_ref[...], w_ref[...], preferred_element_type=jnp.float32)
    h = h + b_ref[...]
    h = 0.5 * h * (1.0 + lax.erf(h * _INV_SQRT2))
    o_ref[...] = _ln(h, g_ref[...], be_ref[...])


def _tc_pre_body(h_ref, wl_ref, bl_ref, wr_ref, br_ref, xl_ref, xr_ref):
    h = h_ref[...]
    xl_ref[...] = jnp.dot(h, wl_ref[...], preferred_element_type=jnp.float32) + bl_ref[...]
    xr_ref[...] = jnp.dot(h, wr_ref[...], preferred_element_type=jnp.float32) + br_ref[...]


def _tc_post_body(h_ref, m_ref, bo_ref, g_ref, b_ref, o_ref):
    o_ref[...] = _ln(h_ref[...] + m_ref[...] + bo_ref[...], g_ref[...], b_ref[...])


def _tc_final_body(h_ref, wg_ref, bg_ref, g_ref, b_ref, o_ref):
    h = h_ref[...]
    z = jnp.dot(h, wg_ref[...], preferred_element_type=jnp.float32) + bg_ref[...]
    gate = 1.0 / (1.0 + jnp.exp(-z))
    o_ref[...] = _ln(h * gate, g_ref[...], b_ref[...])


def _row_spec(d):
    return pl.BlockSpec((R, d), lambda i: (i, 0))


def _full_spec(shape):
    return pl.BlockSpec(shape, lambda i: tuple(0 for _ in shape))


def _tc_input(x, w, b, g, be):
    return pl.pallas_call(
        _tc_input_body,
        grid=(N // R,),
        in_specs=[_row_spec(IN_DIM), _full_spec((IN_DIM, HID)),
                  _full_spec((1, HID)), _full_spec((1, HID)), _full_spec((1, HID))],
        out_specs=_row_spec(HID),
        out_shape=jax.ShapeDtypeStruct((N, HID), jnp.float32),
    )(x, w, b, g, be)


def _tc_pre(h, wl, bl, wr, br):
    return pl.pallas_call(
        _tc_pre_body,
        grid=(N // R,),
        in_specs=[_row_spec(HID), _full_spec((HID, HID)), _full_spec((1, HID)),
                  _full_spec((HID, HID)), _full_spec((1, HID))],
        out_specs=[_row_spec(HID), _row_spec(HID)],
        out_shape=[jax.ShapeDtypeStruct((N, HID), jnp.float32),
                   jax.ShapeDtypeStruct((N, HID), jnp.float32)],
    )(h, wl, bl, wr, br)


def _tc_post(h, m, bo, g, b):
    return pl.pallas_call(
        _tc_post_body,
        grid=(N // R,),
        in_specs=[_row_spec(HID), _row_spec(HID), _full_spec((1, HID)),
                  _full_spec((1, HID)), _full_spec((1, HID))],
        out_specs=_row_spec(HID),
        out_shape=jax.ShapeDtypeStruct((N, HID), jnp.float32),
    )(h, m, bo, g, b)


def _tc_final(h, wg, bg, g, b):
    return pl.pallas_call(
        _tc_final_body,
        grid=(N // R,),
        in_specs=[_row_spec(HID), _full_spec((HID, HID)), _full_spec((1, HID)),
                  _full_spec((1, HID)), _full_spec((1, HID))],
        out_specs=_row_spec(HID),
        out_shape=jax.ShapeDtypeStruct((N, HID), jnp.float32),
    )(h, wg, bg, g, b)


_mesh = plsc.VectorSubcoreMesh(core_axis_name="c", subcore_axis_name="s")

_GDN = lax.GatherDimensionNumbers(
    offset_dims=(), collapsed_slice_dims=(0,), start_index_map=(0,))


def _shuf(x, idx):
    return lax.gather(x, idx[:, None], _GDN, (1,),
                      mode=lax.GatherScatterMode.PROMISE_IN_BOUNDS)


@functools.partial(
    pl.kernel,
    out_type=jax.ShapeDtypeStruct((NPAD * HID,), jnp.float32),
    mesh=_mesh,
    compiler_params=pltpu.CompilerParams(use_tc_tiling_on_sc=False),
    scratch_types=[
        pltpu.VMEM((C,), jnp.int32),
        pltpu.VMEM((C + 16,), jnp.int32),
        pltpu.VMEM((C,), jnp.float32),
        pltpu.VMEM((C,), jnp.float32),
        pltpu.VMEM((C, HID), jnp.float32),
        pltpu.VMEM(((DPR + 1) * HID,), jnp.float32),
        pltpu.VMEM(((DPR + 1) * HID,), jnp.float32),
        pltpu.VMEM(((DPR + 1) * 16,), jnp.float32),
        pltpu.VMEM((2, HID), jnp.float32),
        pltpu.VMEM((16, 16), jnp.float32),
        pltpu.VMEM((NR + 16,), jnp.int32),
        pltpu.VMEM((NR + 16,), jnp.int32),
        pltpu.SemaphoreType.DMA,
    ],
)
def _sc_gat(xl_hbm, xr_hbm, src_hbm, dst_hbm, ea0_hbm, ea1_hbm, barr_hbm,
            earr_hbm, we_hbm, att_hbm, out_hbm,
            idxs_v, idxd_v, ea0_v, ea1_v, xl_v, xrt_v, num_v, den_v,
            we_v, att_v, barr_v, earr_v, sem1):
    wid = lax.axis_index("s") * 2 + lax.axis_index("c")

    pltpu.sync_copy(barr_hbm, barr_v.at[pl.ds(0, NR)])
    pltpu.sync_copy(earr_hbm, earr_v.at[pl.ds(0, NR)])
    pltpu.sync_copy(we_hbm, we_v)
    pltpu.sync_copy(att_hbm, att_v)

    lane = lax.iota(jnp.int32, 16)
    zero16 = jnp.zeros((16,), jnp.float32)

    we0 = [we_v[0, pl.ds(16 * h, 16)] for h in range(HEADS)]
    we1 = [we_v[1, pl.ds(16 * h, 16)] for h in range(HEADS)]
    att_r = [att_v[h] for h in range(HEADS)]
    bitrev = (((lane & 1) << 3) | ((lane & 2) << 1)
              | ((lane & 4) >> 1) | ((lane & 8) >> 3))

    def _combine(a, b, d):
        m = (lane & d) == 0
        return jnp.where(m, a, b) + _shuf(jnp.where(m, b, a), lane ^ d)

    for ph in range(2):
        rid = wid * 2 + ph
        base = rid * DPR

        pltpu.sync_copy(
            xr_hbm.at[pl.ds(pl.multiple_of(base * HID, 64), DPR * HID)],
            xrt_v.at[pl.ds(0, DPR * HID)])

        b_w = barr_v[pl.ds(rid, 16)][0]
        e_w = earr_v[pl.ds(rid, 16)][0]
        start0 = b_w & jnp.int32(-C)
        nch = jnp.right_shift(e_w - start0 + jnp.int32(C - 1), 6)

        def _zero_num(i, carry):
            num_v[pl.ds(pl.multiple_of(i * 16, 16), 16)] = zero16
            return carry

        lax.fori_loop(0, (DPR + 1) * HID // 16, _zero_num, 0)

        def _zero_den(i, carry):
            den_v[pl.ds(pl.multiple_of(i * 16, 16), 16)] = zero16
            return carry

        lax.fori_loop(0, DPR + 1, _zero_den, 0)

        def _edge(e, ecarry):
            g16 = pl.multiple_of((e >> 4) << 4, 16)
            jj = jnp.full((16,), e & 15, jnp.int32)
            d = idxd_v[pl.ds(e, 16)][0]
            own = (d >= base) & (d < base + DPR)
            dl = jnp.where(own, d - base, jnp.int32(DPR))
            a0v = _shuf(ea0_v[pl.ds(g16, 16)], jj)
            a1v = _shuf(ea1_v[pl.ds(g16, 16)], jj)
            rb = dl * HID
            xls = []
            vecs = []
            for h in range(HEADS):
                xlh = xl_v[e, pl.ds(16 * h, 16)]
                xrh = xrt_v[pl.ds(pl.multiple_of(rb + 16 * h, 16), 16)]
                sh = xlh + xrh + a0v * we0[h] + a1v * we1[h]
                sh = jnp.where(sh >= 0.0, sh, NEG_SLOPE * sh)
                vecs.append(sh * att_r[h])
                xls.append(xlh)
            vecs += [zero16] * (16 - HEADS)
            for dd in (8, 4, 2, 1):
                vecs = [_combine(vecs[i], vecs[i + 1], dd)
                        for i in range(0, len(vecs), 2)]
            av = jnp.exp(_shuf(vecs[0], bitrev))
            for h in range(HEADS):
                avh = _shuf(av, jnp.full((16,), h, jnp.int32))
                plsc.addupdate(
                    num_v.at[pl.ds(pl.multiple_of(rb + 16 * h, 16), 16)],
                    avh * xls[h])
            plsc.addupdate(den_v.at[pl.ds(pl.multiple_of(dl * 16, 16), 16)], av)
            return ecarry

        @pl.loop(0, nch)
        def _chunk(k):
            s = pl.multiple_of(start0 + k * C, C)
            pltpu.sync_copy(src_hbm.at[pl.ds(s, C)], idxs_v)
            pltpu.sync_copy(dst_hbm.at[pl.ds(s, C)], idxd_v.at[pl.ds(0, C)])
            pltpu.sync_copy(ea0_hbm.at[pl.ds(s, C)], ea0_v)
            pltpu.sync_copy(ea1_hbm.at[pl.ds(s, C)], ea1_v)
            gl = pltpu.async_copy(xl_hbm.at[idxs_v], xl_v, sem1)
            gl.wait()
            lax.fori_loop(0, C, _edge, 0)

        def _fin(d, carry):
            dv = den_v[pl.ds(pl.multiple_of(d * 16, 16), 16)]
            inv = 1.0 / (dv + 1e-16)
            for h in range(HEADS):
                off = pl.multiple_of(d * HID + 16 * h, 16)
                num_v[pl.ds(off, 16)] = num_v[pl.ds(off, 16)] * _shuf(
                    inv, jnp.full((16,), h, jnp.int32))
            return carry

        lax.fori_loop(0, DPR, _fin, 0)

        pltpu.sync_copy(
            num_v.at[pl.ds(0, DPR * HID)],
            out_hbm.at[pl.ds(pl.multiple_of(base * HID, 64), DPR * HID)])


def kernel(x, edge_index, edge_attr, params):
    p = params
    src = edge_index[0]
    dst = edge_index[1]

    order = jnp.argsort(dst)
    src_s = jnp.take(src, order)
    dst_s = jnp.take(dst, order)
    ea_s = jnp.take(edge_attr, order, axis=0)
    src_p = jnp.concatenate([src_s, jnp.zeros((C,), jnp.int32)])
    dst_p = jnp.concatenate([dst_s, jnp.full((C,), NPAD, jnp.int32)])
    ea0_p = jnp.concatenate([ea_s[:, 0], jnp.zeros((C,), jnp.float32)])
    ea1_p = jnp.concatenate([ea_s[:, 1], jnp.zeros((C,), jnp.float32)])
    marks = jnp.arange(0, NPAD + 1, DPR, dtype=jnp.int32)
    bnd = jnp.searchsorted(dst_s, marks).astype(jnp.int32)
    barr = bnd[:NR]
    earr = bnd[1:NR + 1]

    def row(v):
        return v.reshape(1, HID)

    h = _tc_input(x, p['W_in'], row(p['b_in']), row(p['g_in']), row(p['be_in']))
    for l in range(LAYERS):
        xl, xr = _tc_pre(h, p[f'Wl{l}'], row(p[f'bl{l}']), p[f'Wr{l}'], row(p[f'br{l}']))
        xr_flat = jnp.pad(xr, ((0, NPAD - N), (0, 0))).reshape(-1)
        att_pad = jnp.zeros((16, 16), jnp.float32).at[:HEADS].set(p[f'att{l}'])
        msg = _sc_gat(xl, xr_flat, src_p, dst_p, ea0_p, ea1_p, barr, earr,
                      p[f'We{l}'], att_pad)
        msg = msg.reshape(NPAD, HID)[:N]
        h = _tc_post(h, msg, row(p[f'bo{l}']), row(p[f'g{l}']), row(p[f'b{l}']))
    return _tc_final(h, p['Wg'], row(p['bg']), row(p['g_f']), row(p['b_f']))


# speedup vs baseline: 31.7743x; 1.0287x over previous
import functools

import jax
import jax.numpy as jnp
from jax import lax
from jax.experimental import pallas as pl
from jax.experimental.pallas import tpu as pltpu
from jax.experimental.pallas import tpu_sc as plsc

N = 10000
E = 320000
IN_DIM = 128
HID = 192
HEADS = 12
OUT = 16
LAYERS = 3
NEG_SLOPE = 0.2
EPS = 1e-5

NW = 32
NR = 64
DPR = 160
NPAD = NR * DPR
C = 64
E_PAD = E + C

R = 1000
_INV_SQRT2 = 0.7071067811865476


def _ln(h, g, b):
    mu = jnp.mean(h, axis=-1, keepdims=True)
    var = jnp.mean((h - mu) ** 2, axis=-1, keepdims=True)
    return (h - mu) / jnp.sqrt(var + EPS) * g + b


def _tc_input_body(x_ref, w_ref, b_ref, g_ref, be_ref, o_ref):
    h = jnp.dot(x_ref[...], w_ref[...], preferred_element_type=jnp.float32)
    h = h + b_ref[...]
    h = 0.5 * h * (1.0 + lax.erf(h * _INV_SQRT2))
    o_ref[...] = _ln(h, g_ref[...], be_ref[...])


def _tc_pre_body(h_ref, wl_ref, bl_ref, wr_ref, br_ref, xl_ref, xr_ref):
    h = h_ref[...]
    xl_ref[...] = jnp.dot(h, wl_ref[...], preferred_element_type=jnp.float32) + bl_ref[...]
    xr_ref[...] = jnp.dot(h, wr_ref[...], preferred_element_type=jnp.float32) + br_ref[...]


def _tc_post_body(h_ref, m_ref, bo_ref, g_ref, b_ref, o_ref):
    o_ref[...] = _ln(h_ref[...] + m_ref[...] + bo_ref[...], g_ref[...], b_ref[...])


def _tc_final_body(h_ref, wg_ref, bg_ref, g_ref, b_ref, o_ref):
    h = h_ref[...]
    z = jnp.dot(h, wg_ref[...], preferred_element_type=jnp.float32) + bg_ref[...]
    gate = 1.0 / (1.0 + jnp.exp(-z))
    o_ref[...] = _ln(h * gate, g_ref[...], b_ref[...])


def _row_spec(d):
    return pl.BlockSpec((R, d), lambda i: (i, 0))


def _full_spec(shape):
    return pl.BlockSpec(shape, lambda i: tuple(0 for _ in shape))


def _tc_input(x, w, b, g, be):
    return pl.pallas_call(
        _tc_input_body,
        grid=(N // R,),
        in_specs=[_row_spec(IN_DIM), _full_spec((IN_DIM, HID)),
                  _full_spec((1, HID)), _full_spec((1, HID)), _full_spec((1, HID))],
        out_specs=_row_spec(HID),
        out_shape=jax.ShapeDtypeStruct((N, HID), jnp.float32),
    )(x, w, b, g, be)


def _tc_pre(h, wl, bl, wr, br):
    return pl.pallas_call(
        _tc_pre_body,
        grid=(N // R,),
        in_specs=[_row_spec(HID), _full_spec((HID, HID)), _full_spec((1, HID)),
                  _full_spec((HID, HID)), _full_spec((1, HID))],
        out_specs=[_row_spec(HID), _row_spec(HID)],
        out_shape=[jax.ShapeDtypeStruct((N, HID), jnp.float32),
                   jax.ShapeDtypeStruct((N, HID), jnp.float32)],
    )(h, wl, bl, wr, br)


def _tc_post(h, m, bo, g, b):
    return pl.pallas_call(
        _tc_post_body,
        grid=(N // R,),
        in_specs=[_row_spec(HID), _row_spec(HID), _full_spec((1, HID)),
                  _full_spec((1, HID)), _full_spec((1, HID))],
        out_specs=_row_spec(HID),
        out_shape=jax.ShapeDtypeStruct((N, HID), jnp.float32),
    )(h, m, bo, g, b)


def _tc_final(h, wg, bg, g, b):
    return pl.pallas_call(
        _tc_final_body,
        grid=(N // R,),
        in_specs=[_row_spec(HID), _full_spec((HID, HID)), _full_spec((1, HID)),
                  _full_spec((1, HID)), _full_spec((1, HID))],
        out_specs=_row_spec(HID),
        out_shape=jax.ShapeDtypeStruct((N, HID), jnp.float32),
    )(h, wg, bg, g, b)


_mesh = plsc.VectorSubcoreMesh(core_axis_name="c", subcore_axis_name="s")

_GDN = lax.GatherDimensionNumbers(
    offset_dims=(), collapsed_slice_dims=(0,), start_index_map=(0,))


def _shuf(x, idx):
    return lax.gather(x, idx[:, None], _GDN, (1,),
                      mode=lax.GatherScatterMode.PROMISE_IN_BOUNDS)


@functools.partial(
    pl.kernel,
    out_type=jax.ShapeDtypeStruct((NPAD * HID,), jnp.float32),
    mesh=_mesh,
    compiler_params=pltpu.CompilerParams(use_tc_tiling_on_sc=False),
    scratch_types=[
        pltpu.VMEM((C,), jnp.int32),
        pltpu.VMEM((C + 16,), jnp.int32),
        pltpu.VMEM((C,), jnp.float32),
        pltpu.VMEM((C,), jnp.float32),
        pltpu.VMEM((C, HID), jnp.float32),
        pltpu.VMEM(((DPR + 1) * HID,), jnp.float32),
        pltpu.VMEM(((DPR + 1) * HID,), jnp.float32),
        pltpu.VMEM(((DPR + 1) * 16,), jnp.float32),
        pltpu.VMEM((2, HID), jnp.float32),
        pltpu.VMEM((16, 16), jnp.float32),
        pltpu.VMEM((NR + 16,), jnp.int32),
        pltpu.VMEM((NR + 16,), jnp.int32),
        pltpu.SemaphoreType.DMA,
    ],
)
def _sc_gat(xl_hbm, xr_hbm, src_hbm, dst_hbm, ea0_hbm, ea1_hbm, barr_hbm,
            earr_hbm, we_hbm, att_hbm, out_hbm,
            idxs_v, idxd_v, ea0_v, ea1_v, xl_v, xrt_v, num_v, den_v,
            we_v, att_v, barr_v, earr_v, sem1):
    wid = lax.axis_index("s") * 2 + lax.axis_index("c")

    pltpu.sync_copy(barr_hbm, barr_v.at[pl.ds(0, NR)])
    pltpu.sync_copy(earr_hbm, earr_v.at[pl.ds(0, NR)])
    pltpu.sync_copy(we_hbm, we_v)
    pltpu.sync_copy(att_hbm, att_v)

    lane = lax.iota(jnp.int32, 16)
    zero16 = jnp.zeros((16,), jnp.float32)

    we0 = [we_v[0, pl.ds(16 * h, 16)] for h in range(HEADS)]
    we1 = [we_v[1, pl.ds(16 * h, 16)] for h in range(HEADS)]
    att_r = [att_v[h] for h in range(HEADS)]
    bitrev = (((lane & 1) << 3) | ((lane & 2) << 1)
              | ((lane & 4) >> 1) | ((lane & 8) >> 3))

    def _combine(a, b, d):
        m = (lane & d) == 0
        return jnp.where(m, a, b) + _shuf(jnp.where(m, b, a), lane ^ d)

    for ph in range(2):
        rid = wid * 2 + ph
        base = rid * DPR

        pltpu.sync_copy(
            xr_hbm.at[pl.ds(pl.multiple_of(base * HID, 64), DPR * HID)],
            xrt_v.at[pl.ds(0, DPR * HID)])

        b_w = barr_v[pl.ds(rid, 16)][0]
        e_w = earr_v[pl.ds(rid, 16)][0]
        start0 = b_w & jnp.int32(-C)
        nch = jnp.right_shift(e_w - start0 + jnp.int32(C - 1), 6)

        def _zero_num(i, carry):
            num_v[pl.ds(pl.multiple_of(i * 16, 16), 16)] = zero16
            return carry

        lax.fori_loop(0, (DPR + 1) * HID // 16, _zero_num, 0)

        def _zero_den(i, carry):
            den_v[pl.ds(pl.multiple_of(i * 16, 16), 16)] = zero16
            return carry

        lax.fori_loop(0, DPR + 1, _zero_den, 0)

        def _edge(e):
            g16 = pl.multiple_of((e >> 4) << 4, 16)
            jj = jnp.full((16,), e & 15, jnp.int32)
            d = idxd_v[pl.ds(e, 16)][0]
            own = (d >= base) & (d < base + DPR)
            dl = jnp.where(own, d - base, jnp.int32(DPR))
            a0v = _shuf(ea0_v[pl.ds(g16, 16)], jj)
            a1v = _shuf(ea1_v[pl.ds(g16, 16)], jj)
            rb = dl * HID
            xls = []
            vecs = []
            for h in range(HEADS):
                xlh = xl_v[e, pl.ds(16 * h, 16)]
                xrh = xrt_v[pl.ds(pl.multiple_of(rb + 16 * h, 16), 16)]
                sh = xlh + xrh + a0v * we0[h] + a1v * we1[h]
                sh = jnp.where(sh >= 0.0, sh, NEG_SLOPE * sh)
                vecs.append(sh * att_r[h])
                xls.append(xlh)
            vecs += [zero16] * (16 - HEADS)
            for dd in (8, 4, 2, 1):
                vecs = [_combine(vecs[i], vecs[i + 1], dd)
                        for i in range(0, len(vecs), 2)]
            av = jnp.exp(_shuf(vecs[0], bitrev))
            for h in range(HEADS):
                avh = _shuf(av, jnp.full((16,), h, jnp.int32))
                plsc.addupdate(
                    num_v.at[pl.ds(pl.multiple_of(rb + 16 * h, 16), 16)],
                    avh * xls[h])
            plsc.addupdate(den_v.at[pl.ds(pl.multiple_of(dl * 16, 16), 16)], av)

        def _pair(i, ecarry):
            _edge(2 * i)
            _edge(2 * i + 1)
            return ecarry

        @pl.loop(0, nch)
        def _chunk(k):
            s = pl.multiple_of(start0 + k * C, C)
            pltpu.sync_copy(src_hbm.at[pl.ds(s, C)], idxs_v)
            pltpu.sync_copy(dst_hbm.at[pl.ds(s, C)], idxd_v.at[pl.ds(0, C)])
            pltpu.sync_copy(ea0_hbm.at[pl.ds(s, C)], ea0_v)
            pltpu.sync_copy(ea1_hbm.at[pl.ds(s, C)], ea1_v)
            gl = pltpu.async_copy(xl_hbm.at[idxs_v], xl_v, sem1)
            gl.wait()
            lax.fori_loop(0, C // 2, _pair, 0)

        def _fin(d, carry):
            dv = den_v[pl.ds(pl.multiple_of(d * 16, 16), 16)]
            inv = 1.0 / (dv + 1e-16)
            for h in range(HEADS):
                off = pl.multiple_of(d * HID + 16 * h, 16)
                num_v[pl.ds(off, 16)] = num_v[pl.ds(off, 16)] * _shuf(
                    inv, jnp.full((16,), h, jnp.int32))
            return carry

        lax.fori_loop(0, DPR, _fin, 0)

        pltpu.sync_copy(
            num_v.at[pl.ds(0, DPR * HID)],
            out_hbm.at[pl.ds(pl.multiple_of(base * HID, 64), DPR * HID)])


def kernel(x, edge_index, edge_attr, params):
    p = params
    src = edge_index[0]
    dst = edge_index[1]

    order = jnp.argsort(dst)
    src_s = jnp.take(src, order)
    dst_s = jnp.take(dst, order)
    ea_s = jnp.take(edge_attr, order, axis=0)
    src_p = jnp.concatenate([src_s, jnp.zeros((C,), jnp.int32)])
    dst_p = jnp.concatenate([dst_s, jnp.full((C,), NPAD, jnp.int32)])
    ea0_p = jnp.concatenate([ea_s[:, 0], jnp.zeros((C,), jnp.float32)])
    ea1_p = jnp.concatenate([ea_s[:, 1], jnp.zeros((C,), jnp.float32)])
    marks = jnp.arange(0, NPAD + 1, DPR, dtype=jnp.int32)
    bnd = jnp.searchsorted(dst_s, marks).astype(jnp.int32)
    barr = bnd[:NR]
    earr = bnd[1:NR + 1]

    def row(v):
        return v.reshape(1, HID)

    h = _tc_input(x, p['W_in'], row(p['b_in']), row(p['g_in']), row(p['be_in']))
    for l in range(LAYERS):
        xl, xr = _tc_pre(h, p[f'Wl{l}'], row(p[f'bl{l}']), p[f'Wr{l}'], row(p[f'br{l}']))
        xr_flat = jnp.pad(xr, ((0, NPAD - N), (0, 0))).reshape(-1)
        att_pad = jnp.zeros((16, 16), jnp.float32).at[:HEADS].set(p[f'att{l}'])
        msg = _sc_gat(xl, xr_flat, src_p, dst_p, ea0_p, ea1_p, barr, earr,
                      p[f'We{l}'], att_pad)
        msg = msg.reshape(NPAD, HID)[:N]
        h = _tc_post(h, msg, row(p[f'bo{l}']), row(p[f'g{l}']), row(p[f'b{l}']))
    return _tc_final(h, p['Wg'], row(p['bg']), row(p['g_f']), row(p['b_f']))


# speedup vs baseline: 37.7200x; 1.1871x over previous
import functools

import jax
import jax.numpy as jnp
from jax import lax
from jax.experimental import pallas as pl
from jax.experimental.pallas import tpu as pltpu
from jax.experimental.pallas import tpu_sc as plsc

N = 10000
E = 320000
IN_DIM = 128
HID = 192
HEADS = 12
OUT = 16
LAYERS = 3
NEG_SLOPE = 0.2
EPS = 1e-5

NW = 32
NR = 64
DPR = 160
NPAD = NR * DPR
C = 64
E_PAD = E + 4 * C

R = 1000
_INV_SQRT2 = 0.7071067811865476


def _ln(h, g, b):
    mu = jnp.mean(h, axis=-1, keepdims=True)
    var = jnp.mean((h - mu) ** 2, axis=-1, keepdims=True)
    return (h - mu) / jnp.sqrt(var + EPS) * g + b


def _tc_input_body(x_ref, w_ref, b_ref, g_ref, be_ref, o_ref):
    h = jnp.dot(x_ref[...], w_ref[...], preferred_element_type=jnp.float32)
    h = h + b_ref[...]
    h = 0.5 * h * (1.0 + lax.erf(h * _INV_SQRT2))
    o_ref[...] = _ln(h, g_ref[...], be_ref[...])


def _tc_pre_body(h_ref, wl_ref, bl_ref, wr_ref, br_ref, xl_ref, xr_ref):
    h = h_ref[...]
    xl_ref[...] = jnp.dot(h, wl_ref[...], preferred_element_type=jnp.float32) + bl_ref[...]
    xr_ref[...] = jnp.dot(h, wr_ref[...], preferred_element_type=jnp.float32) + br_ref[...]


def _tc_post_body(h_ref, m_ref, bo_ref, g_ref, b_ref, o_ref):
    o_ref[...] = _ln(h_ref[...] + m_ref[...] + bo_ref[...], g_ref[...], b_ref[...])


def _tc_final_body(h_ref, wg_ref, bg_ref, g_ref, b_ref, o_ref):
    h = h_ref[...]
    z = jnp.dot(h, wg_ref[...], preferred_element_type=jnp.float32) + bg_ref[...]
    gate = 1.0 / (1.0 + jnp.exp(-z))
    o_ref[...] = _ln(h * gate, g_ref[...], b_ref[...])


def _row_spec(d):
    return pl.BlockSpec((R, d), lambda i: (i, 0))


def _full_spec(shape):
    return pl.BlockSpec(shape, lambda i: tuple(0 for _ in shape))


def _tc_input(x, w, b, g, be):
    return pl.pallas_call(
        _tc_input_body,
        grid=(N // R,),
        in_specs=[_row_spec(IN_DIM), _full_spec((IN_DIM, HID)),
                  _full_spec((1, HID)), _full_spec((1, HID)), _full_spec((1, HID))],
        out_specs=_row_spec(HID),
        out_shape=jax.ShapeDtypeStruct((N, HID), jnp.float32),
    )(x, w, b, g, be)


def _tc_pre(h, wl, bl, wr, br):
    return pl.pallas_call(
        _tc_pre_body,
        grid=(N // R,),
        in_specs=[_row_spec(HID), _full_spec((HID, HID)), _full_spec((1, HID)),
                  _full_spec((HID, HID)), _full_spec((1, HID))],
        out_specs=[_row_spec(HID), _row_spec(HID)],
        out_shape=[jax.ShapeDtypeStruct((N, HID), jnp.float32),
                   jax.ShapeDtypeStruct((N, HID), jnp.float32)],
    )(h, wl, bl, wr, br)


def _tc_post(h, m, bo, g, b):
    return pl.pallas_call(
        _tc_post_body,
        grid=(N // R,),
        in_specs=[_row_spec(HID), _row_spec(HID), _full_spec((1, HID)),
                  _full_spec((1, HID)), _full_spec((1, HID))],
        out_specs=_row_spec(HID),
        out_shape=jax.ShapeDtypeStruct((N, HID), jnp.float32),
    )(h, m, bo, g, b)


def _tc_final(h, wg, bg, g, b):
    return pl.pallas_call(
        _tc_final_body,
        grid=(N // R,),
        in_specs=[_row_spec(HID), _full_spec((HID, HID)), _full_spec((1, HID)),
                  _full_spec((1, HID)), _full_spec((1, HID))],
        out_specs=_row_spec(HID),
        out_shape=jax.ShapeDtypeStruct((N, HID), jnp.float32),
    )(h, wg, bg, g, b)


_mesh = plsc.VectorSubcoreMesh(core_axis_name="c", subcore_axis_name="s")

_GDN = lax.GatherDimensionNumbers(
    offset_dims=(), collapsed_slice_dims=(0,), start_index_map=(0,))


def _shuf(x, idx):
    return lax.gather(x, idx[:, None], _GDN, (1,),
                      mode=lax.GatherScatterMode.PROMISE_IN_BOUNDS)


@functools.partial(
    pl.kernel,
    out_type=jax.ShapeDtypeStruct((NPAD * HID,), jnp.float32),
    mesh=_mesh,
    compiler_params=pltpu.CompilerParams(use_tc_tiling_on_sc=False),
    scratch_types=[
        pltpu.VMEM((2 * C + 16,), jnp.int32),
        pltpu.VMEM((2 * C + 16,), jnp.int32),
        pltpu.VMEM((2 * C,), jnp.float32),
        pltpu.VMEM((2 * C,), jnp.float32),
        pltpu.VMEM((C, HID), jnp.float32),
        pltpu.VMEM((C, HID), jnp.float32),
        pltpu.VMEM(((DPR + 1) * HID,), jnp.float32),
        pltpu.VMEM(((DPR + 1) * HID,), jnp.float32),
        pltpu.VMEM(((DPR + 1) * 16,), jnp.float32),
        pltpu.VMEM((2, HID), jnp.float32),
        pltpu.VMEM((16, 16), jnp.float32),
        pltpu.VMEM((NR + 16,), jnp.int32),
        pltpu.VMEM((NR + 16,), jnp.int32),
        pltpu.SemaphoreType.DMA,
        pltpu.SemaphoreType.DMA,
    ],
)
def _sc_gat(xl_hbm, xr_hbm, ii_hbm, ea_hbm, barr_hbm,
            earr_hbm, we_hbm, att_hbm, out_hbm,
            iia_v, iib_v, eaa_v, eab_v, xla_v, xlb_v, xrt_v, num_v, den_v,
            we_v, att_v, barr_v, earr_v, sema, semb):
    wid = lax.axis_index("s") * 2 + lax.axis_index("c")

    pltpu.sync_copy(barr_hbm, barr_v.at[pl.ds(0, NR)])
    pltpu.sync_copy(earr_hbm, earr_v.at[pl.ds(0, NR)])
    pltpu.sync_copy(we_hbm, we_v)
    pltpu.sync_copy(att_hbm, att_v)

    lane = lax.iota(jnp.int32, 16)
    zero16 = jnp.zeros((16,), jnp.float32)

    we0 = [we_v[0, pl.ds(16 * h, 16)] for h in range(HEADS)]
    we1 = [we_v[1, pl.ds(16 * h, 16)] for h in range(HEADS)]
    att_r = [att_v[h] for h in range(HEADS)]
    bitrev = (((lane & 1) << 3) | ((lane & 2) << 1)
              | ((lane & 4) >> 1) | ((lane & 8) >> 3))

    def _combine(a, b, d):
        m = (lane & d) == 0
        return jnp.where(m, a, b) + _shuf(jnp.where(m, b, a), lane ^ d)

    for ph in range(2):
        rid = wid * 2 + ph
        base = rid * DPR

        pltpu.sync_copy(
            xr_hbm.at[pl.ds(pl.multiple_of(base * HID, 64), DPR * HID)],
            xrt_v.at[pl.ds(0, DPR * HID)])

        b_w = barr_v[pl.ds(rid, 16)][0]
        e_w = earr_v[pl.ds(rid, 16)][0]
        start0 = b_w & jnp.int32(-C)
        npairs = jnp.right_shift(e_w - start0 + jnp.int32(2 * C - 1), 7)

        def _zero_num(i, carry):
            num_v[pl.ds(pl.multiple_of(i * 16, 16), 16)] = zero16
            return carry

        lax.fori_loop(0, (DPR + 1) * HID // 16, _zero_num, 0)

        def _zero_den(i, carry):
            den_v[pl.ds(pl.multiple_of(i * 16, 16), 16)] = zero16
            return carry

        lax.fori_loop(0, DPR + 1, _zero_den, 0)

        def _edge(e, ii_v, ea_v, xl_v):
            g16 = pl.multiple_of((e >> 4) << 4, 16)
            jj = jnp.full((16,), e & 15, jnp.int32)
            d = ii_v[pl.ds(C + e, 16)][0]
            own = (d >= base) & (d < base + DPR)
            dl = jnp.where(own, d - base, jnp.int32(DPR))
            a0v = _shuf(ea_v[pl.ds(g16, 16)], jj)
            a1v = _shuf(ea_v[pl.ds(C + g16, 16)], jj)
            rb = dl * HID
            xls = []
            vecs = []
            for h in range(HEADS):
                xlh = xl_v[e, pl.ds(16 * h, 16)]
                xrh = xrt_v[pl.ds(pl.multiple_of(rb + 16 * h, 16), 16)]
                sh = xlh + xrh + a0v * we0[h] + a1v * we1[h]
                sh = jnp.where(sh >= 0.0, sh, NEG_SLOPE * sh)
                vecs.append(sh * att_r[h])
                xls.append(xlh)
            vecs += [zero16] * (16 - HEADS)
            for dd in (8, 4, 2, 1):
                vecs = [_combine(vecs[i], vecs[i + 1], dd)
                        for i in range(0, len(vecs), 2)]
            av = jnp.exp(_shuf(vecs[0], bitrev))
            for h in range(HEADS):
                avh = _shuf(av, jnp.full((16,), h, jnp.int32))
                plsc.addupdate(
                    num_v.at[pl.ds(pl.multiple_of(rb + 16 * h, 16), 16)],
                    avh * xls[h])
            plsc.addupdate(den_v.at[pl.ds(pl.multiple_of(dl * 16, 16), 16)], av)

        def _compute(ii_v, ea_v, xl_v):
            def _pair(i, ecarry):
                _edge(2 * i, ii_v, ea_v, xl_v)
                _edge(2 * i + 1, ii_v, ea_v, xl_v)
                return ecarry

            lax.fori_loop(0, C // 2, _pair, 0)

        def _load(s2, ii_v, ea_v, xl_v, sem):
            pltpu.sync_copy(ii_hbm.at[pl.ds(s2, 2 * C)], ii_v.at[pl.ds(0, 2 * C)])
            pltpu.sync_copy(ea_hbm.at[pl.ds(s2, 2 * C)], ea_v)
            pltpu.async_copy(xl_hbm.at[ii_v.at[pl.ds(0, C)]], xl_v, sem)

        def _drain(ii_v, xl_v, sem):
            pltpu.make_async_copy(
                xl_hbm.at[ii_v.at[pl.ds(0, C)]], xl_v, sem).wait()

        s0 = pl.multiple_of(start0 * 2, 2 * C)
        _load(s0, iia_v, eaa_v, xla_v, sema)

        @pl.loop(0, npairs)
        def _chunkpair(j):
            s2 = pl.multiple_of(start0 * 2 + j * 4 * C, 2 * C)
            _load(s2 + 2 * C, iib_v, eab_v, xlb_v, semb)
            _drain(iia_v, xla_v, sema)
            _compute(iia_v, eaa_v, xla_v)
            _load(s2 + 4 * C, iia_v, eaa_v, xla_v, sema)
            _drain(iib_v, xlb_v, semb)
            _compute(iib_v, eab_v, xlb_v)

        _drain(iia_v, xla_v, sema)

        def _fin(d, carry):
            dv = den_v[pl.ds(pl.multiple_of(d * 16, 16), 16)]
            inv = 1.0 / (dv + 1e-16)
            for h in range(HEADS):
                off = pl.multiple_of(d * HID + 16 * h, 16)
                num_v[pl.ds(off, 16)] = num_v[pl.ds(off, 16)] * _shuf(
                    inv, jnp.full((16,), h, jnp.int32))
            return carry

        lax.fori_loop(0, DPR, _fin, 0)

        pltpu.sync_copy(
            num_v.at[pl.ds(0, DPR * HID)],
            out_hbm.at[pl.ds(pl.multiple_of(base * HID, 64), DPR * HID)])


def kernel(x, edge_index, edge_attr, params):
    p = params
    src = edge_index[0]
    dst = edge_index[1]

    order = jnp.argsort(dst)
    src_s = jnp.take(src, order)
    dst_s = jnp.take(dst, order)
    ea_s = jnp.take(edge_attr, order, axis=0)
    pad = E_PAD - E
    src_p = jnp.concatenate([src_s, jnp.zeros((pad,), jnp.int32)])
    dst_p = jnp.concatenate([dst_s, jnp.full((pad,), NPAD, jnp.int32)])
    ea0_p = jnp.concatenate([ea_s[:, 0], jnp.zeros((pad,), jnp.float32)])
    ea1_p = jnp.concatenate([ea_s[:, 1], jnp.zeros((pad,), jnp.float32)])
    ii_pack = jnp.stack(
        [src_p.reshape(E_PAD // C, C), dst_p.reshape(E_PAD // C, C)],
        axis=1).reshape(-1)
    ea_pack = jnp.stack(
        [ea0_p.reshape(E_PAD // C, C), ea1_p.reshape(E_PAD // C, C)],
        axis=1).reshape(-1)
    marks = jnp.arange(0, NPAD + 1, DPR, dtype=jnp.int32)
    bnd = jnp.searchsorted(dst_s, marks).astype(jnp.int32)
    barr = bnd[:NR]
    earr = bnd[1:NR + 1]

    def row(v):
        return v.reshape(1, HID)

    h = _tc_input(x, p['W_in'], row(p['b_in']), row(p['g_in']), row(p['be_in']))
    for l in range(LAYERS):
        xl, xr = _tc_pre(h, p[f'Wl{l}'], row(p[f'bl{l}']), p[f'Wr{l}'], row(p[f'br{l}']))
        xr_flat = jnp.pad(xr, ((0, NPAD - N), (0, 0))).reshape(-1)
        att_pad = jnp.zeros((16, 16), jnp.float32).at[:HEADS].set(p[f'att{l}'])
        msg = _sc_gat(xl, xr_flat, ii_pack, ea_pack, barr, earr,
                      p[f'We{l}'], att_pad)
        msg = msg.reshape(NPAD, HID)[:N]
        h = _tc_post(h, msg, row(p[f'bo{l}']), row(p[f'g{l}']), row(p[f'b{l}']))
    return _tc_final(h, p['Wg'], row(p['bg']), row(p['g_f']), row(p['b_f']))
